# baseline (speedup 1.0000x reference)
import functools

import jax
import jax.numpy as jnp
from jax import lax
from jax.experimental import pallas as pl
from jax.experimental.pallas import tpu as pltpu
from jax.experimental.pallas import tpu_sc as plsc

B = 16384
D = 64
MARGIN = 1.0
NC = 2
NS = 16
NW = NC * NS
PER_W = B // NW
CHUNK = 128
NCHUNK = PER_W // CHUNK
GROUPS = CHUNK // 16
UNROLL = 4


def _sc_body(ent_hbm, rel_hbm, il_hbm, ir_hbm, irel_hbm, inl_hbm, inr_hbm,
             o1_hbm, o2_hbm, o3_hbm,
             rawl_v, rawr_v, rawrel_v, rawnl_v, rawnr_v,
             pl_v, pr_v, prel_v, pnl_v, pnr_v,
             hl_v, hr_v, hrel_v, hnl_v, hnr_v,
             l_v, r_v, rl_v, nl_v, nr_v,
             s1_v, s2_v, s3_v, sem, semi):
    wid = lax.axis_index("c") * NS + lax.axis_index("s")
    base = wid * PER_W
    iota16 = lax.iota(jnp.int32, 16)

    idx_ins = (il_hbm, ir_hbm, irel_hbm, inl_hbm, inr_hbm)
    raw_bufs = (rawl_v, rawr_v, rawrel_v, rawnl_v, rawnr_v)
    pair_bufs = (pl_v, pr_v, prel_v, pnl_v, pnr_v)
    half_bufs = (hl_v, hr_v, hrel_v, hnl_v, hnr_v)
    row_bufs = (l_v, r_v, rl_v, nl_v, nr_v)
    tables = (ent_hbm, ent_hbm, rel_hbm, ent_hbm, ent_hbm)
    flat_bufs = tuple(b.reshape(1, CHUNK * 2 * D) for b in row_bufs)
    zvec = jnp.zeros((16,), jnp.int32)

    for c in range(NCHUNK):
        off = base + c * CHUNK
        ih = [pltpu.async_copy(idx_ins[t].at[pl.ds(off, CHUNK)],
                               raw_bufs[t], semi) for t in range(5)]
        for h in ih:
            h.wait()
        for t in range(5):
            for g in range(GROUPS):
                raw = raw_bufs[t][pl.ds(g * 16, 16)]
                pair_bufs[t][pl.ds(g * 16, 16)] = lax.shift_right_logical(raw, 1)
                half_bufs[t][pl.ds(g * 16, 16)] = (raw & 1) * D

        gh = [pltpu.async_copy(tables[t].at[pair_bufs[t]], row_bufs[t], sem)
              for t in range(5)]
        for h in gh:
            h.wait()

        for g in range(GROUPS):
            rowbase = lax.shift_left(iota16 + (g * 16), 7)
            zero = jnp.zeros((16,), jnp.float32)
            gb = [rowbase + hb[pl.ds(g * 16, 16)] for hb in half_bufs]

            def body(k, accs, gb=gb, zvec=zvec):
                a1, a2, a3 = accs
                dd0 = k * UNROLL
                for u in range(UNROLL):
                    dd = dd0 + u
                    le = plsc.load_gather(flat_bufs[0], [zvec, gb[0] + dd])
                    ri = plsc.load_gather(flat_bufs[1], [zvec, gb[1] + dd])
                    re = plsc.load_gather(flat_bufs[2], [zvec, gb[2] + dd])
                    nl = plsc.load_gather(flat_bufs[3], [zvec, gb[3] + dd])
                    nr = plsc.load_gather(flat_bufs[4], [zvec, gb[4] + dd])
                    a = le + re
                    t1 = a - ri
                    t2 = (nl + re) - ri
                    t3 = a - nr
                    a1 = a1 + t1 * t1
                    a2 = a2 + t2 * t2
                    a3 = a3 + t3 * t3
                return (a1, a2, a3)

            acc1, acc2, acc3 = lax.fori_loop(0, D // UNROLL, body,
                                             (zero, zero, zero))
            pos = c * CHUNK + g * 16
            s1_v[pl.ds(pos, 16)] = acc1
            s2_v[pl.ds(pos, 16)] = acc2
            s3_v[pl.ds(pos, 16)] = acc3

    pltpu.sync_copy(s1_v, o1_hbm.at[pl.ds(base, PER_W)])
    pltpu.sync_copy(s2_v, o2_hbm.at[pl.ds(base, PER_W)])
    pltpu.sync_copy(s3_v, o3_hbm.at[pl.ds(base, PER_W)])


_sc_kernel = functools.partial(
    pl.kernel,
    mesh=plsc.VectorSubcoreMesh(core_axis_name="c", subcore_axis_name="s",
                                num_cores=NC, num_subcores=NS),
    out_type=[jax.ShapeDtypeStruct((B,), jnp.float32)] * 3,
    scratch_types=(
        [pltpu.VMEM((CHUNK,), jnp.int32)] * 15
        + [pltpu.VMEM((CHUNK, 2 * D), jnp.float32)] * 5
        + [pltpu.VMEM((PER_W,), jnp.float32)] * 3
        + [pltpu.SemaphoreType.DMA] * 2
    ),
    compiler_params=pltpu.CompilerParams(needs_layout_passes=False,
                                         use_tc_tiling_on_sc=True),
)(_sc_body)


def _tc_body(p_ref, n1_ref, n2_ref, o_ref):
    p = jnp.sqrt(p_ref[...])
    n1 = jnp.sqrt(n1_ref[...])
    n2 = jnp.sqrt(n2_ref[...])
    c1 = p - n1 + MARGIN
    c2 = p - n2 + MARGIN
    cost = c1 * (c1 > 0) + c2 * (c2 > 0)
    o_ref[0, 0] = jnp.sum(cost) * (1.0 / B)


def kernel(entity_table, relation_table, leftEnIndices, rightEnIndices,
           relIndices, negLeftEnIndices, negRightEnIndices):
    il = leftEnIndices.astype(jnp.int32)
    ir = rightEnIndices.astype(jnp.int32)
    irel = relIndices.astype(jnp.int32)
    inl = negLeftEnIndices.astype(jnp.int32)
    inr = negRightEnIndices.astype(jnp.int32)

    ent128 = entity_table.reshape(500000, 2 * D)
    rel128 = relation_table.reshape(500, 2 * D)

    s1, s2, s3 = _sc_kernel(ent128, rel128, il, ir, irel, inl, inr)

    out = pl.pallas_call(
        _tc_body,
        out_shape=jax.ShapeDtypeStruct((1, 1), jnp.float32),
        out_specs=pl.BlockSpec(memory_space=pltpu.SMEM),
    )(s1.reshape(128, 128), s2.reshape(128, 128), s3.reshape(128, 128))
    return out[0, 0]


# speedup vs baseline: 1.0301x; 1.0301x over previous
import functools

import jax
import jax.numpy as jnp
from jax import lax
from jax.experimental import pallas as pl
from jax.experimental.pallas import tpu as pltpu
from jax.experimental.pallas import tpu_sc as plsc

B = 16384
D = 64
MARGIN = 1.0
NC = 2
NS = 16
NW = NC * NS
PER_W = B // NW
CHUNK = 64
NCHUNK = PER_W // CHUNK
GROUPS = CHUNK // 16
UNROLL = 4
NBUF = 2


def _sc_body(ent_hbm, rel_hbm, il_hbm, ir_hbm, irel_hbm, inl_hbm, inr_hbm,
             o1_hbm, o2_hbm, o3_hbm,
             *scratch):
    raw_bufs = scratch[0:10]
    pair_bufs = scratch[10:20]
    half_bufs = scratch[20:30]
    row_bufs = scratch[30:40]
    s1_v, s2_v, s3_v = scratch[40:43]
    semi = scratch[43:45]
    sem = scratch[45:47]

    wid = lax.axis_index("c") * NS + lax.axis_index("s")
    base = wid * PER_W
    iota16 = lax.iota(jnp.int32, 16)

    idx_ins = (il_hbm, ir_hbm, irel_hbm, inl_hbm, inr_hbm)
    tables = (ent_hbm, ent_hbm, rel_hbm, ent_hbm, ent_hbm)

    def issue_idx(c, b):
        off = base + c * CHUNK
        for t in range(5):
            pltpu.async_copy(idx_ins[t].at[pl.ds(off, CHUNK)],
                             raw_bufs[b * 5 + t], semi[b])

    def prep_and_gather(b):
        for t in range(5):
            pltpu.make_async_copy(idx_ins[t].at[pl.ds(0, CHUNK)],
                                  raw_bufs[b * 5 + t], semi[b]).wait()
        for t in range(5):
            for g in range(GROUPS):
                raw = raw_bufs[b * 5 + t][pl.ds(g * 16, 16)]
                pair_bufs[b * 5 + t][pl.ds(g * 16, 16)] = (
                    lax.shift_right_logical(raw, 1))
                half_bufs[b * 5 + t][pl.ds(g * 16, 16)] = (raw & 1) * D
        for t in range(5):
            pltpu.async_copy(tables[t].at[pair_bufs[b * 5 + t]],
                             row_bufs[b * 5 + t], sem[b])

    def compute(c, b):
        for t in range(5):
            pltpu.make_async_copy(tables[t].at[pl.ds(0, CHUNK), :],
                                  row_bufs[b * 5 + t], sem[b]).wait()
        bufs = row_bufs[b * 5:b * 5 + 5]
        for g in range(GROUPS):
            rvec = iota16 + (g * 16)
            zero = jnp.zeros((16,), jnp.float32)
            ho = [half_bufs[b * 5 + t][pl.ds(g * 16, 16)] for t in range(5)]

            def body(k, accs, rvec=rvec, ho=ho, bufs=bufs):
                a1, a2, a3 = accs
                dd0 = k * UNROLL
                for u in range(UNROLL):
                    dd = dd0 + u
                    le = plsc.load_gather(bufs[0], [rvec, ho[0] + dd])
                    ri = plsc.load_gather(bufs[1], [rvec, ho[1] + dd])
                    re = plsc.load_gather(bufs[2], [rvec, ho[2] + dd])
                    nl = plsc.load_gather(bufs[3], [rvec, ho[3] + dd])
                    nr = plsc.load_gather(bufs[4], [rvec, ho[4] + dd])
                    a = le + re
                    t1 = a - ri
                    t2 = (nl + re) - ri
                    t3 = a - nr
                    a1 = a1 + t1 * t1
                    a2 = a2 + t2 * t2
                    a3 = a3 + t3 * t3
                return (a1, a2, a3)

            acc1, acc2, acc3 = lax.fori_loop(0, D // UNROLL, body,
                                             (zero, zero, zero))
            pos = c * CHUNK + g * 16
            s1_v[pl.ds(pos, 16)] = acc1
            s2_v[pl.ds(pos, 16)] = acc2
            s3_v[pl.ds(pos, 16)] = acc3

    issue_idx(0, 0)
    prep_and_gather(0)
    for c in range(NCHUNK):
        b = c % NBUF
        if c + 1 < NCHUNK:
            nb = (c + 1) % NBUF
            issue_idx(c + 1, nb)
            prep_and_gather(nb)
        compute(c, b)

    pltpu.sync_copy(s1_v, o1_hbm.at[pl.ds(base, PER_W)])
    pltpu.sync_copy(s2_v, o2_hbm.at[pl.ds(base, PER_W)])
    pltpu.sync_copy(s3_v, o3_hbm.at[pl.ds(base, PER_W)])


_sc_kernel = functools.partial(
    pl.kernel,
    mesh=plsc.VectorSubcoreMesh(core_axis_name="c", subcore_axis_name="s",
                                num_cores=NC, num_subcores=NS),
    out_type=[jax.ShapeDtypeStruct((B,), jnp.float32)] * 3,
    scratch_types=(
        [pltpu.VMEM((CHUNK,), jnp.int32)] * 30
        + [pltpu.VMEM((CHUNK, 2 * D), jnp.float32)] * 10
        + [pltpu.VMEM((PER_W,), jnp.float32)] * 3
        + [pltpu.SemaphoreType.DMA] * 4
    ),
    compiler_params=pltpu.CompilerParams(needs_layout_passes=False,
                                         use_tc_tiling_on_sc=True),
)(_sc_body)


def _tc_body(p_ref, n1_ref, n2_ref, o_ref):
    p = jnp.sqrt(p_ref[...])
    n1 = jnp.sqrt(n1_ref[...])
    n2 = jnp.sqrt(n2_ref[...])
    c1 = p - n1 + MARGIN
    c2 = p - n2 + MARGIN
    cost = c1 * (c1 > 0) + c2 * (c2 > 0)
    o_ref[0, 0] = jnp.sum(cost) * (1.0 / B)


def kernel(entity_table, relation_table, leftEnIndices, rightEnIndices,
           relIndices, negLeftEnIndices, negRightEnIndices):
    il = leftEnIndices.astype(jnp.int32)
    ir = rightEnIndices.astype(jnp.int32)
    irel = relIndices.astype(jnp.int32)
    inl = negLeftEnIndices.astype(jnp.int32)
    inr = negRightEnIndices.astype(jnp.int32)

    ent128 = entity_table.reshape(500000, 2 * D)
    rel128 = relation_table.reshape(500, 2 * D)

    s1, s2, s3 = _sc_kernel(ent128, rel128, il, ir, irel, inl, inr)

    out = pl.pallas_call(
        _tc_body,
        out_shape=jax.ShapeDtypeStruct((1, 1), jnp.float32),
        out_specs=pl.BlockSpec(memory_space=pltpu.SMEM),
    )(s1.reshape(128, 128), s2.reshape(128, 128), s3.reshape(128, 128))
    return out[0, 0]
